# baseline (speedup 1.0000x reference)
import jax
import jax.numpy as jnp
from jax import lax
from jax.experimental import pallas as pl
from jax.experimental.pallas import tpu as pltpu

_B, _T, _D, _H, _K = 64, 576, 768, 32, 1024
_N = _B * _T
_R = 4096
_G = _N // _R


_C = 4
_RC = _R // _C


def _chain(x, wp, bp, wb, bb, e, en, etr):
    s = lax.dot_general(x, wp, (((1,), (1,)), ((), ())),
                        preferred_element_type=jnp.float32)
    s = s + bp
    dist = en - 2.0 * lax.dot_general(s, e, (((1,), (0,)), ((), ())),
                                      preferred_element_type=jnp.float32)
    ind = jnp.argmin(dist, axis=1)
    lo = ind & 127
    hi = ind >> 7
    onehot_lo = (lax.broadcasted_iota(jnp.int32, (_RC, 128), 1)
                 == lo[:, None]).astype(jnp.float32)
    cand = lax.dot_general(onehot_lo, etr, (((1,), (0,)), ((), ())),
                           preferred_element_type=jnp.float32)
    msk = (lax.broadcasted_iota(jnp.int32, (_RC, 256), 1) >> 5) == hi[:, None]
    qsel = jnp.where(msk, cand, 0.0)
    q = (qsel[:, 0:32] + qsel[:, 32:64] + qsel[:, 64:96] + qsel[:, 96:128]
         + qsel[:, 128:160] + qsel[:, 160:192] + qsel[:, 192:224]
         + qsel[:, 224:256])
    out = lax.dot_general(q, wb, (((1,), (1,)), ((), ())),
                          preferred_element_type=jnp.float32) + bb
    d = q - s
    return out, jnp.sum(d * d)


def _body(x_ref, wp_ref, bp_ref, wb_ref, bb_ref, e_ref, etr_ref,
          out_ref, loss_ref):
    i = pl.program_id(0)
    e = e_ref[...]
    en = jnp.sum(e * e, axis=0, keepdims=True)
    wp, bp, wb, bb, etr = (wp_ref[...], bp_ref[...], wb_ref[...],
                           bb_ref[...], etr_ref[...])
    part = jnp.float32(0.0)
    for c in range(_C):
        rows = pl.ds(c * _RC, _RC)
        out_c, p_c = _chain(x_ref[rows, :], wp, bp, wb, bb, e, en, etr)
        out_ref[rows, :] = out_c
        part = part + p_c

    @pl.when(i == 0)
    def _init():
        loss_ref[...] = jnp.zeros_like(loss_ref)

    loss_ref[...] += part


def kernel(state, W_proj, b_proj, W_back, b_back, embed):
    x2d = state.reshape(_N, _D)
    etr = embed.T.reshape(8, 128, _H).transpose(1, 0, 2).reshape(128, 8 * _H)
    out2d, loss_sum = pl.pallas_call(
        _body,
        grid=(_G,),
        in_specs=[
            pl.BlockSpec((_R, _D), lambda i: (i, 0)),
            pl.BlockSpec((_H, _D), lambda i: (0, 0)),
            pl.BlockSpec((1, _H), lambda i: (0, 0)),
            pl.BlockSpec((_D, _H), lambda i: (0, 0)),
            pl.BlockSpec((1, _D), lambda i: (0, 0)),
            pl.BlockSpec((_H, _K), lambda i: (0, 0)),
            pl.BlockSpec((128, 8 * _H), lambda i: (0, 0)),
        ],
        out_specs=[
            pl.BlockSpec((_R, _D), lambda i: (i, 0)),
            pl.BlockSpec((1, 1), lambda i: (0, 0)),
        ],
        out_shape=[
            jax.ShapeDtypeStruct((_N, _D), jnp.float32),
            jax.ShapeDtypeStruct((1, 1), jnp.float32),
        ],
        compiler_params=pltpu.CompilerParams(
            dimension_semantics=("arbitrary",),
        ),
    )(x2d, W_proj, b_proj.reshape(1, _H), W_back, b_back.reshape(1, _D),
      embed, etr)
    out = out2d.reshape(_B, _T, _D)
    extra_loss = loss_sum[0, 0] / jnp.float32(_N * _H)
    att_scores = jnp.zeros((1, 1, 10), dtype=jnp.float32)
    return out, extra_loss, att_scores


# speedup vs baseline: 1.0037x; 1.0037x over previous
import jax
import jax.numpy as jnp
from jax import lax
from jax.experimental import pallas as pl
from jax.experimental.pallas import tpu as pltpu

_B, _T, _D, _H, _K = 64, 576, 768, 32, 1024
_N = _B * _T
_R = 2048
_G = _N // _R


_C = 2
_RC = _R // _C


def _chain(x, wp, bp, wb, bb, e, en, etr):
    s = lax.dot_general(x, wp, (((1,), (1,)), ((), ())),
                        preferred_element_type=jnp.float32)
    s = s + bp
    dist = en - 2.0 * lax.dot_general(s, e, (((1,), (0,)), ((), ())),
                                      preferred_element_type=jnp.float32)
    ind = jnp.argmin(dist, axis=1)
    lo = ind & 127
    hi = ind >> 7
    onehot_lo = (lax.broadcasted_iota(jnp.int32, (_RC, 128), 1)
                 == lo[:, None]).astype(jnp.float32)
    cand = lax.dot_general(onehot_lo, etr, (((1,), (0,)), ((), ())),
                           preferred_element_type=jnp.float32)
    msk = (lax.broadcasted_iota(jnp.int32, (_RC, 256), 1) >> 5) == hi[:, None]
    qsel = jnp.where(msk, cand, 0.0)
    q = (qsel[:, 0:32] + qsel[:, 32:64] + qsel[:, 64:96] + qsel[:, 96:128]
         + qsel[:, 128:160] + qsel[:, 160:192] + qsel[:, 192:224]
         + qsel[:, 224:256])
    out = lax.dot_general(q, wb, (((1,), (1,)), ((), ())),
                          preferred_element_type=jnp.float32) + bb
    d = q - s
    return out, jnp.sum(d * d)


def _body(x_ref, wp_ref, bp_ref, wb_ref, bb_ref, e_ref, etr_ref,
          out_ref, loss_ref):
    i = pl.program_id(0)
    e = e_ref[...]
    en = jnp.sum(e * e, axis=0, keepdims=True)
    wp, bp, wb, bb, etr = (wp_ref[...], bp_ref[...], wb_ref[...],
                           bb_ref[...], etr_ref[...])
    part = jnp.float32(0.0)
    for c in range(_C):
        rows = pl.ds(c * _RC, _RC)
        out_c, p_c = _chain(x_ref[rows, :], wp, bp, wb, bb, e, en, etr)
        out_ref[rows, :] = out_c
        part = part + p_c

    @pl.when(i == 0)
    def _init():
        loss_ref[...] = jnp.zeros_like(loss_ref)

    loss_ref[...] += part


def kernel(state, W_proj, b_proj, W_back, b_back, embed):
    x2d = state.reshape(_N, _D)
    etr = embed.T.reshape(8, 128, _H).transpose(1, 0, 2).reshape(128, 8 * _H)
    out2d, loss_sum = pl.pallas_call(
        _body,
        grid=(_G,),
        in_specs=[
            pl.BlockSpec((_R, _D), lambda i: (i, 0)),
            pl.BlockSpec((_H, _D), lambda i: (0, 0)),
            pl.BlockSpec((1, _H), lambda i: (0, 0)),
            pl.BlockSpec((_D, _H), lambda i: (0, 0)),
            pl.BlockSpec((1, _D), lambda i: (0, 0)),
            pl.BlockSpec((_H, _K), lambda i: (0, 0)),
            pl.BlockSpec((128, 8 * _H), lambda i: (0, 0)),
        ],
        out_specs=[
            pl.BlockSpec((_R, _D), lambda i: (i, 0)),
            pl.BlockSpec((1, 1), lambda i: (0, 0)),
        ],
        out_shape=[
            jax.ShapeDtypeStruct((_N, _D), jnp.float32),
            jax.ShapeDtypeStruct((1, 1), jnp.float32),
        ],
        compiler_params=pltpu.CompilerParams(
            dimension_semantics=("arbitrary",),
        ),
    )(x2d, W_proj, b_proj.reshape(1, _H), W_back, b_back.reshape(1, _D),
      embed, etr)
    out = out2d.reshape(_B, _T, _D)
    extra_loss = loss_sum[0, 0] / jnp.float32(_N * _H)
    att_scores = jnp.zeros((1, 1, 10), dtype=jnp.float32)
    return out, extra_loss, att_scores


# speedup vs baseline: 1.0646x; 1.0606x over previous
import jax
import jax.numpy as jnp
from jax import lax
from jax.experimental import pallas as pl
from jax.experimental.pallas import tpu as pltpu

_B, _T, _D, _H, _K = 64, 576, 768, 32, 1024
_N = _B * _T
_R = 2048
_G = _N // _R
_C = 2
_RC = _R // _C


def _front(x, wp, bp, e, en, etr):
    s = lax.dot_general(x, wp, (((1,), (1,)), ((), ())),
                        preferred_element_type=jnp.float32)
    s = s + bp
    dist = en - 2.0 * lax.dot_general(s, e, (((1,), (0,)), ((), ())),
                                      preferred_element_type=jnp.float32)
    ind = jnp.argmin(dist, axis=1)
    lo = ind & 127
    hi = ind >> 7
    onehot_lo = (lax.broadcasted_iota(jnp.int32, (_RC, 128), 1)
                 == lo[:, None]).astype(jnp.float32)
    cand = lax.dot_general(onehot_lo, etr, (((1,), (0,)), ((), ())),
                           preferred_element_type=jnp.float32)
    msk = (lax.broadcasted_iota(jnp.int32, (_RC, 256), 1) >> 5) == hi[:, None]
    qsel = jnp.where(msk, cand, 0.0)
    q = (qsel[:, 0:32] + qsel[:, 32:64] + qsel[:, 64:96] + qsel[:, 96:128]
         + qsel[:, 128:160] + qsel[:, 160:192] + qsel[:, 192:224]
         + qsel[:, 224:256])
    d = q - s
    return q, jnp.sum(d * d)


def _body(x_ref, wp_ref, bp_ref, wb_ref, bb_ref, e_ref, etr_ref,
          out_ref, loss_ref, qs_ref):
    i = pl.program_id(0)

    @pl.when(i == 0)
    def _init():
        loss_ref[...] = jnp.zeros_like(loss_ref)

    e = e_ref[...]
    en = jnp.sum(e * e, axis=0, keepdims=True)
    wp, bp, etr = wp_ref[...], bp_ref[...], etr_ref[...]
    part = jnp.float32(0.0)
    for c in range(_C):
        rows = pl.ds(c * _RC, _RC)
        q_c, p_c = _front(x_ref[rows, :], wp, bp, e, en, etr)
        qs_ref[i % 2, rows, :] = q_c
        part = part + p_c
    loss_ref[...] += jnp.where(i < _G, part, 0.0)

    q = qs_ref[(i + 1) % 2]
    out = lax.dot_general(q, wb_ref[...], (((1,), (1,)), ((), ())),
                          preferred_element_type=jnp.float32)
    out_ref[...] = out + bb_ref[...]


def kernel(state, W_proj, b_proj, W_back, b_back, embed):
    x2d = state.reshape(_N, _D)
    etr = embed.T.reshape(8, 128, _H).transpose(1, 0, 2).reshape(128, 8 * _H)
    out2d, loss_sum = pl.pallas_call(
        _body,
        grid=(_G + 1,),
        in_specs=[
            pl.BlockSpec((_R, _D), lambda i: (jnp.minimum(i, _G - 1), 0)),
            pl.BlockSpec((_H, _D), lambda i: (0, 0)),
            pl.BlockSpec((1, _H), lambda i: (0, 0)),
            pl.BlockSpec((_D, _H), lambda i: (0, 0)),
            pl.BlockSpec((1, _D), lambda i: (0, 0)),
            pl.BlockSpec((_H, _K), lambda i: (0, 0)),
            pl.BlockSpec((128, 8 * _H), lambda i: (0, 0)),
        ],
        out_specs=[
            pl.BlockSpec((_R, _D), lambda i: (jnp.maximum(i - 1, 0), 0)),
            pl.BlockSpec((1, 1), lambda i: (0, 0)),
        ],
        out_shape=[
            jax.ShapeDtypeStruct((_N, _D), jnp.float32),
            jax.ShapeDtypeStruct((1, 1), jnp.float32),
        ],
        scratch_shapes=[pltpu.VMEM((2, _R, _H), jnp.float32)],
        compiler_params=pltpu.CompilerParams(
            dimension_semantics=("arbitrary",),
        ),
    )(x2d, W_proj, b_proj.reshape(1, _H), W_back, b_back.reshape(1, _D),
      embed, etr)
    out = out2d.reshape(_B, _T, _D)
    extra_loss = loss_sum[0, 0] / jnp.float32(_N * _H)
    att_scores = jnp.zeros((1, 1, 10), dtype=jnp.float32)
    return out, extra_loss, att_scores


# speedup vs baseline: 1.0939x; 1.0275x over previous
import jax
import jax.numpy as jnp
from jax import lax
from jax.experimental import pallas as pl
from jax.experimental.pallas import tpu as pltpu

_B, _T, _D, _H, _K = 64, 576, 768, 32, 1024
_N = _B * _T
_R = 2048
_G = _N // _R
_C = 4
_RC = _R // _C


def _front(x, wp, bp, e, en, etr):
    s = lax.dot_general(x, wp, (((1,), (1,)), ((), ())),
                        preferred_element_type=jnp.float32)
    s = s + bp
    dist = en - 2.0 * lax.dot_general(s, e, (((1,), (0,)), ((), ())),
                                      preferred_element_type=jnp.float32)
    ind = jnp.argmin(dist, axis=1)
    lo = ind & 127
    hi = ind >> 7
    onehot_lo = (lax.broadcasted_iota(jnp.int32, (_RC, 128), 1)
                 == lo[:, None]).astype(jnp.float32)
    cand = lax.dot_general(onehot_lo, etr, (((1,), (0,)), ((), ())),
                           preferred_element_type=jnp.float32)
    msk = (lax.broadcasted_iota(jnp.int32, (_RC, 256), 1) >> 5) == hi[:, None]
    qsel = jnp.where(msk, cand, 0.0)
    q = (qsel[:, 0:32] + qsel[:, 32:64] + qsel[:, 64:96] + qsel[:, 96:128]
         + qsel[:, 128:160] + qsel[:, 160:192] + qsel[:, 192:224]
         + qsel[:, 224:256])
    d = q - s
    return q, jnp.sum(d * d)


def _body(x_ref, wp_ref, bp_ref, wb_ref, bb_ref, e_ref, etr_ref,
          out_ref, loss_ref, qs_ref):
    i = pl.program_id(0)

    @pl.when(i == 0)
    def _init():
        loss_ref[...] = jnp.zeros_like(loss_ref)

    e = e_ref[...]
    en = jnp.sum(e * e, axis=0, keepdims=True)
    wp, bp, etr = wp_ref[...], bp_ref[...], etr_ref[...]
    part = jnp.float32(0.0)
    for c in range(_C):
        rows = pl.ds(c * _RC, _RC)
        q_c, p_c = _front(x_ref[rows, :], wp, bp, e, en, etr)
        qs_ref[i % 2, rows, :] = q_c
        part = part + p_c
    loss_ref[...] += jnp.where(i < _G, part, 0.0)

    q = qs_ref[(i + 1) % 2]
    out = lax.dot_general(q, wb_ref[...], (((1,), (1,)), ((), ())),
                          preferred_element_type=jnp.float32)
    out_ref[...] = out + bb_ref[...]


def kernel(state, W_proj, b_proj, W_back, b_back, embed):
    x2d = state.reshape(_N, _D)
    etr = embed.T.reshape(8, 128, _H).transpose(1, 0, 2).reshape(128, 8 * _H)
    out2d, loss_sum = pl.pallas_call(
        _body,
        grid=(_G + 1,),
        in_specs=[
            pl.BlockSpec((_R, _D), lambda i: (jnp.minimum(i, _G - 1), 0)),
            pl.BlockSpec((_H, _D), lambda i: (0, 0)),
            pl.BlockSpec((1, _H), lambda i: (0, 0)),
            pl.BlockSpec((_D, _H), lambda i: (0, 0)),
            pl.BlockSpec((1, _D), lambda i: (0, 0)),
            pl.BlockSpec((_H, _K), lambda i: (0, 0)),
            pl.BlockSpec((128, 8 * _H), lambda i: (0, 0)),
        ],
        out_specs=[
            pl.BlockSpec((_R, _D), lambda i: (jnp.maximum(i - 1, 0), 0)),
            pl.BlockSpec((1, 1), lambda i: (0, 0)),
        ],
        out_shape=[
            jax.ShapeDtypeStruct((_N, _D), jnp.float32),
            jax.ShapeDtypeStruct((1, 1), jnp.float32),
        ],
        scratch_shapes=[pltpu.VMEM((2, _R, _H), jnp.float32)],
        compiler_params=pltpu.CompilerParams(
            dimension_semantics=("arbitrary",),
        ),
    )(x2d, W_proj, b_proj.reshape(1, _H), W_back, b_back.reshape(1, _D),
      embed, etr)
    out = out2d.reshape(_B, _T, _D)
    extra_loss = loss_sum[0, 0] / jnp.float32(_N * _H)
    att_scores = jnp.zeros((1, 1, 10), dtype=jnp.float32)
    return out, extra_loss, att_scores
